# Initial kernel scaffold; baseline (speedup 1.0000x reference)
import jax
import jax.numpy as jnp
from jax.experimental import pallas as pl


def kernel(x, adj, W1, b1, W2, b2):
    raise NotImplementedError("write your pallas kernel here")


# speedup vs baseline: 1.0671x; 1.0671x over previous
import functools

import jax
import jax.numpy as jnp
from jax.experimental import pallas as pl
from jax.experimental.pallas import tpu as pltpu

_N = 10000
_C = 10
_F = 128
_H = 128
_K = 8
_ALPHA = 0.1
_BM = 400
_BME = 1000


def _encoder_body(x_ref, w1_ref, b1_ref, w2_ref, b2_ref, z_ref):
    h = jax.lax.dot_general(
        x_ref[...], w1_ref[...], (((1,), (1,)), ((), ())),
        preferred_element_type=jnp.float32)
    h = jax.nn.relu(h + b1_ref[...])
    z = jax.lax.dot_general(
        h, w2_ref[...], (((1,), (1,)), ((), ())),
        preferred_element_type=jnp.float32)
    z_ref[...] = z + b2_ref[...]


def _prop_body(adj_ref, cur_ref, z_ref, out_ref, *, last):
    acc = jax.lax.dot_general(
        adj_ref[...], cur_ref[...], (((1,), (0,)), ((), ())),
        preferred_element_type=jnp.float32)
    new = acc * (1.0 - _ALPHA) + _ALPHA * z_ref[...]
    if last:
        out_ref[...] = jax.nn.log_softmax(new, axis=1)
    else:
        out_ref[...] = new


def _encode(x, W1, b1, W2, b2):
    return pl.pallas_call(
        _encoder_body,
        grid=(_N // _BME,),
        in_specs=[
            pl.BlockSpec((_BME, _F), lambda r: (r, 0)),
            pl.BlockSpec((_H, _F), lambda r: (0, 0)),
            pl.BlockSpec((1, _H), lambda r: (0, 0)),
            pl.BlockSpec((_C, _H), lambda r: (0, 0)),
            pl.BlockSpec((1, _C), lambda r: (0, 0)),
        ],
        out_specs=pl.BlockSpec((_BME, _C), lambda r: (r, 0)),
        out_shape=jax.ShapeDtypeStruct((_N, _C), jnp.float32),
        compiler_params=pltpu.CompilerParams(
            dimension_semantics=("parallel",)),
    )(x, W1, b1, W2, b2)


def _prop_step(adj, cur, z, last):
    body = functools.partial(_prop_body, last=last)
    return pl.pallas_call(
        body,
        grid=(_N // _BM,),
        in_specs=[
            pl.BlockSpec((_BM, _N), lambda r: (r, 0)),
            pl.BlockSpec((_N, _C), lambda r: (0, 0)),
            pl.BlockSpec((_BM, _C), lambda r: (r, 0)),
        ],
        out_specs=pl.BlockSpec((_BM, _C), lambda r: (r, 0)),
        out_shape=jax.ShapeDtypeStruct((_N, _C), jnp.float32),
        compiler_params=pltpu.CompilerParams(
            dimension_semantics=("parallel",)),
    )(adj, cur, z)


def kernel(x, adj, W1, b1, W2, b2):
    z = _encode(x, W1, b1.reshape(1, _H), W2, b2.reshape(1, _C))
    cur = z
    for k in range(_K):
        cur = _prop_step(adj, cur, z, last=(k == _K - 1))
    return cur


# speedup vs baseline: 1.1569x; 1.0841x over previous
import jax
import jax.numpy as jnp
from jax.experimental import pallas as pl
from jax.experimental.pallas import tpu as pltpu

_N = 10000
_C = 10
_F = 128
_H = 128
_K = 8
_ALPHA = 0.1
_BM = 400
_BME = 1000


def _encoder_body(x_ref, w1_ref, b1_ref, w2_ref, b2_ref, z_ref):
    h = jax.lax.dot_general(
        x_ref[...], w1_ref[...], (((1,), (1,)), ((), ())),
        preferred_element_type=jnp.float32)
    h = jax.nn.relu(h + b1_ref[...])
    z = jax.lax.dot_general(
        h, w2_ref[...], (((1,), (1,)), ((), ())),
        preferred_element_type=jnp.float32)
    z_ref[...] = z + b2_ref[...]


def _encode(x, W1, b1, W2, b2):
    return pl.pallas_call(
        _encoder_body,
        grid=(_N // _BME,),
        in_specs=[
            pl.BlockSpec((_BME, _F), lambda r: (r, 0)),
            pl.BlockSpec((_H, _F), lambda r: (0, 0)),
            pl.BlockSpec((1, _H), lambda r: (0, 0)),
            pl.BlockSpec((_C, _H), lambda r: (0, 0)),
            pl.BlockSpec((1, _C), lambda r: (0, 0)),
        ],
        out_specs=pl.BlockSpec((_BME, _C), lambda r: (r, 0)),
        out_shape=jax.ShapeDtypeStruct((_N, _C), jnp.float32),
        compiler_params=pltpu.CompilerParams(
            dimension_semantics=("parallel",)),
    )(x, W1, b1, W2, b2)


def _prop_body(adj_ref, z_ref, out_ref, cur_ref):
    k = pl.program_id(0)
    r = pl.program_id(1)
    src = jnp.where(k == 0, z_ref[...], cur_ref[(k + 1) % 2])
    acc = jax.lax.dot_general(
        adj_ref[...], src, (((1,), (0,)), ((), ())),
        preferred_element_type=jnp.float32)
    new = acc * (1.0 - _ALPHA) + _ALPHA * z_ref[pl.ds(r * _BM, _BM), :]
    cur_ref[k % 2, pl.ds(r * _BM, _BM), :] = new

    @pl.when(k == _K - 1)
    def _():
        out_ref[...] = jax.nn.log_softmax(new, axis=1)


def _propagate(adj, z):
    return pl.pallas_call(
        _prop_body,
        grid=(_K, _N // _BM),
        in_specs=[
            pl.BlockSpec((_BM, _N), lambda k, r: (r, 0)),
            pl.BlockSpec((_N, _C), lambda k, r: (0, 0)),
        ],
        out_specs=pl.BlockSpec((_BM, _C), lambda k, r: (r, 0)),
        out_shape=jax.ShapeDtypeStruct((_N, _C), jnp.float32),
        scratch_shapes=[pltpu.VMEM((2, _N, _C), jnp.float32)],
        compiler_params=pltpu.CompilerParams(
            dimension_semantics=("arbitrary", "arbitrary")),
    )(adj, z)


def kernel(x, adj, W1, b1, W2, b2):
    z = _encode(x, W1, b1.reshape(1, _H), W2, b2.reshape(1, _C))
    return _propagate(adj, z)
